# baseline (speedup 1.0000x reference)
import jax
import jax.numpy as jnp
from jax.experimental import pallas as pl
from jax.experimental.pallas import tpu as pltpu

_TOP_K = 2
_SMALL = 512
_N_SMALL = 4
_BIG = 2048
_N_EXP = 8


def _top2(logits, idx_ref, w_ref, off):
    lt = logits.T
    n_e = lt.shape[0]
    t = lt.shape[1]
    rows = [lt[e] for e in range(n_e)]
    m1 = rows[0]
    for e in range(1, n_e):
        m1 = jnp.maximum(m1, rows[e])
    i1 = jnp.full_like(m1, n_e - 1, dtype=jnp.int32)
    for e in range(n_e - 2, -1, -1):
        i1 = jnp.where(rows[e] == m1, e, i1)
    neg = jnp.float32(-3.0e38)
    rows2 = [jnp.where(i1 == e, neg, rows[e]) for e in range(n_e)]
    m2 = rows2[0]
    for e in range(1, n_e):
        m2 = jnp.maximum(m2, rows2[e])
    i2 = jnp.full_like(m1, n_e - 1, dtype=jnp.int32)
    for e in range(n_e - 2, -1, -1):
        i2 = jnp.where(rows2[e] == m2, e, i2)
    d = jnp.exp(m2 - m1)
    r = 1.0 / (1.0 + d)
    idx_ref[:, pl.ds(off, t)] = jnp.stack([i1, i2], axis=0)
    w_ref[:, pl.ds(off, t)] = jnp.stack([r, d * r], axis=0)


def _router_step(x_hbm, wt_ref, idx_ref, w_ref, xbuf, sem):
    pid = pl.program_id(0)
    n_steps = pl.num_programs(0)

    def chunk_off(j):
        return jnp.where(j < _N_SMALL, _SMALL * j, _BIG * (j - (_N_SMALL - 1)))

    def start_copy(j, buf):
        @pl.when(j < _N_SMALL)
        def _():
            pltpu.make_async_copy(
                x_hbm.at[pl.ds(chunk_off(j), _SMALL), :],
                xbuf.at[buf, pl.ds(0, _SMALL), :],
                sem.at[buf],
            ).start()

        @pl.when(j >= _N_SMALL)
        def _():
            pltpu.make_async_copy(
                x_hbm.at[pl.ds(chunk_off(j), _BIG), :],
                xbuf.at[buf, pl.ds(0, _BIG), :],
                sem.at[buf],
            ).start()

    @pl.when(pid == 0)
    def _():
        start_copy(0, 0)

    @pl.when(pid + 1 < n_steps)
    def _():
        start_copy(pid + 1, (pid + 1) % 2)

    buf = pid % 2
    off = chunk_off(pid)

    @pl.when(pid < _N_SMALL)
    def _():
        pltpu.make_async_copy(
            x_hbm.at[pl.ds(off, _SMALL), :],
            xbuf.at[buf, pl.ds(0, _SMALL), :],
            sem.at[buf],
        ).wait()
        x = xbuf[buf, 0:_SMALL, :]
        logits = jnp.dot(x, wt_ref[...], preferred_element_type=jnp.float32)
        _top2(logits, idx_ref, w_ref, off)

    @pl.when(pid >= _N_SMALL)
    def _():
        pltpu.make_async_copy(
            x_hbm.at[pl.ds(off, _BIG), :],
            xbuf.at[buf, pl.ds(0, _BIG), :],
            sem.at[buf],
        ).wait()
        x = xbuf[buf, 0:_BIG, :]
        logits = jnp.dot(x, wt_ref[...], preferred_element_type=jnp.float32)
        _top2(logits, idx_ref, w_ref, off)


@jax.jit
def kernel(hidden_states, weight):
    bsz, seq_len, h = hidden_states.shape
    n_tok = bsz * seq_len
    n_exp = weight.shape[0]
    x = hidden_states.reshape(n_tok, h).astype(jnp.float32)
    wt = weight.astype(jnp.float32).T

    n_big = (n_tok - _SMALL * _N_SMALL) // _BIG
    n_steps = _N_SMALL + n_big
    idx_t, w_t = pl.pallas_call(
        _router_step,
        grid=(n_steps,),
        in_specs=[
            pl.BlockSpec(memory_space=pltpu.HBM),
            pl.BlockSpec((h, n_exp), lambda i: (0, 0)),
        ],
        out_specs=[
            pl.BlockSpec((_TOP_K, n_tok), lambda i: (0, 0)),
            pl.BlockSpec((_TOP_K, n_tok), lambda i: (0, 0)),
        ],
        out_shape=[
            jax.ShapeDtypeStruct((_TOP_K, n_tok), jnp.int32),
            jax.ShapeDtypeStruct((_TOP_K, n_tok), jnp.float32),
        ],
        scratch_shapes=[
            pltpu.VMEM((2, _BIG, h), jnp.float32),
            pltpu.SemaphoreType.DMA((2,)),
        ],
        compiler_params=pltpu.CompilerParams(
            dimension_semantics=("arbitrary",),
        ),
    )(x, wt)
    return (idx_t.T, w_t.T)


# speedup vs baseline: 1.0010x; 1.0010x over previous
import jax
import jax.numpy as jnp
from jax.experimental import pallas as pl
from jax.experimental.pallas import tpu as pltpu

_TOP_K = 2
_SMALL = 2048
_N_SMALL = 1
_BIG = 2048
_N_EXP = 8


def _top2(logits, idx_ref, w_ref, off):
    lt = logits.T
    n_e = lt.shape[0]
    t = lt.shape[1]
    rows = [lt[e] for e in range(n_e)]
    m1 = rows[0]
    for e in range(1, n_e):
        m1 = jnp.maximum(m1, rows[e])
    i1 = jnp.full_like(m1, n_e - 1, dtype=jnp.int32)
    for e in range(n_e - 2, -1, -1):
        i1 = jnp.where(rows[e] == m1, e, i1)
    neg = jnp.float32(-3.0e38)
    rows2 = [jnp.where(i1 == e, neg, rows[e]) for e in range(n_e)]
    m2 = rows2[0]
    for e in range(1, n_e):
        m2 = jnp.maximum(m2, rows2[e])
    i2 = jnp.full_like(m1, n_e - 1, dtype=jnp.int32)
    for e in range(n_e - 2, -1, -1):
        i2 = jnp.where(rows2[e] == m2, e, i2)
    d = jnp.exp(m2 - m1)
    r = 1.0 / (1.0 + d)
    idx_ref[:, pl.ds(off, t)] = jnp.stack([i1, i2], axis=0)
    w_ref[:, pl.ds(off, t)] = jnp.stack([r, d * r], axis=0)


def _router_step(x_hbm, wt_ref, idx_ref, w_ref, xbuf, sem):
    pid = pl.program_id(0)
    n_steps = pl.num_programs(0)

    def chunk_off(j):
        return jnp.where(j < _N_SMALL, _SMALL * j, _BIG * (j - (_N_SMALL - 1)))

    def start_copy(j, buf):
        @pl.when(j < _N_SMALL)
        def _():
            pltpu.make_async_copy(
                x_hbm.at[pl.ds(chunk_off(j), _SMALL), :],
                xbuf.at[buf, pl.ds(0, _SMALL), :],
                sem.at[buf],
            ).start()

        @pl.when(j >= _N_SMALL)
        def _():
            pltpu.make_async_copy(
                x_hbm.at[pl.ds(chunk_off(j), _BIG), :],
                xbuf.at[buf, pl.ds(0, _BIG), :],
                sem.at[buf],
            ).start()

    @pl.when(pid == 0)
    def _():
        start_copy(0, 0)

    @pl.when(pid + 1 < n_steps)
    def _():
        start_copy(pid + 1, (pid + 1) % 2)

    buf = pid % 2
    off = chunk_off(pid)

    @pl.when(pid < _N_SMALL)
    def _():
        pltpu.make_async_copy(
            x_hbm.at[pl.ds(off, _SMALL), :],
            xbuf.at[buf, pl.ds(0, _SMALL), :],
            sem.at[buf],
        ).wait()
        x = xbuf[buf, 0:_SMALL, :]
        logits = jnp.dot(x, wt_ref[...], preferred_element_type=jnp.float32)
        _top2(logits, idx_ref, w_ref, off)

    @pl.when(pid >= _N_SMALL)
    def _():
        pltpu.make_async_copy(
            x_hbm.at[pl.ds(off, _BIG), :],
            xbuf.at[buf, pl.ds(0, _BIG), :],
            sem.at[buf],
        ).wait()
        x = xbuf[buf, 0:_BIG, :]
        logits = jnp.dot(x, wt_ref[...], preferred_element_type=jnp.float32)
        _top2(logits, idx_ref, w_ref, off)


@jax.jit
def kernel(hidden_states, weight):
    bsz, seq_len, h = hidden_states.shape
    n_tok = bsz * seq_len
    n_exp = weight.shape[0]
    x = hidden_states.reshape(n_tok, h).astype(jnp.float32)
    wt = weight.astype(jnp.float32).T

    n_big = (n_tok - _SMALL * _N_SMALL) // _BIG
    n_steps = _N_SMALL + n_big
    idx_t, w_t = pl.pallas_call(
        _router_step,
        grid=(n_steps,),
        in_specs=[
            pl.BlockSpec(memory_space=pltpu.HBM),
            pl.BlockSpec((h, n_exp), lambda i: (0, 0)),
        ],
        out_specs=[
            pl.BlockSpec((_TOP_K, n_tok), lambda i: (0, 0)),
            pl.BlockSpec((_TOP_K, n_tok), lambda i: (0, 0)),
        ],
        out_shape=[
            jax.ShapeDtypeStruct((_TOP_K, n_tok), jnp.int32),
            jax.ShapeDtypeStruct((_TOP_K, n_tok), jnp.float32),
        ],
        scratch_shapes=[
            pltpu.VMEM((2, _BIG, h), jnp.float32),
            pltpu.SemaphoreType.DMA((2,)),
        ],
        compiler_params=pltpu.CompilerParams(
            dimension_semantics=("arbitrary",),
        ),
    )(x, wt)
    return (idx_t.T, w_t.T)


# speedup vs baseline: 1.0285x; 1.0274x over previous
import functools

import jax
import jax.numpy as jnp
from jax.experimental import pallas as pl
from jax.experimental.pallas import tpu as pltpu

_TOP_K = 2
_BLOCK_T = 2048


def _router_block(x_ref, wt_ref, idx_ref, w_ref):
    x = x_ref[...]
    wt = wt_ref[...]
    logits = jnp.dot(x, wt, preferred_element_type=jnp.float32)
    lt = logits.T
    n_e = lt.shape[0]
    rows = [lt[e] for e in range(n_e)]

    m1 = rows[0]
    for e in range(1, n_e):
        m1 = jnp.maximum(m1, rows[e])
    i1 = jnp.full_like(m1, n_e - 1, dtype=jnp.int32)
    for e in range(n_e - 2, -1, -1):
        i1 = jnp.where(rows[e] == m1, e, i1)

    neg = jnp.float32(-3.0e38)
    rows2 = [jnp.where(i1 == e, neg, rows[e]) for e in range(n_e)]
    m2 = rows2[0]
    for e in range(1, n_e):
        m2 = jnp.maximum(m2, rows2[e])
    i2 = jnp.full_like(m1, n_e - 1, dtype=jnp.int32)
    for e in range(n_e - 2, -1, -1):
        i2 = jnp.where(rows2[e] == m2, e, i2)

    d = jnp.exp(m2 - m1)
    r = 1.0 / (1.0 + d)
    idx_ref[...] = jnp.stack([i1, i2], axis=0)
    w_ref[...] = jnp.stack([r, d * r], axis=0)


@jax.jit
def kernel(hidden_states, weight):
    bsz, seq_len, h = hidden_states.shape
    n_tok = bsz * seq_len
    n_exp = weight.shape[0]
    x = hidden_states.reshape(n_tok, h).astype(jnp.float32)
    wt = weight.astype(jnp.float32).T

    grid = (n_tok // _BLOCK_T,)
    idx_t, w_t = pl.pallas_call(
        _router_block,
        grid=grid,
        in_specs=[
            pl.BlockSpec((_BLOCK_T, h), lambda i: (i, 0)),
            pl.BlockSpec((h, n_exp), lambda i: (0, 0)),
        ],
        out_specs=[
            pl.BlockSpec((_TOP_K, _BLOCK_T), lambda i: (0, i)),
            pl.BlockSpec((_TOP_K, _BLOCK_T), lambda i: (0, i)),
        ],
        out_shape=[
            jax.ShapeDtypeStruct((_TOP_K, n_tok), jnp.int32),
            jax.ShapeDtypeStruct((_TOP_K, n_tok), jnp.float32),
        ],
        compiler_params=pltpu.CompilerParams(
            dimension_semantics=("arbitrary",),
        ),
    )(x, wt)
    return (idx_t.T, w_t.T)
